# Initial kernel scaffold; baseline (speedup 1.0000x reference)
import jax
import jax.numpy as jnp
from jax.experimental import pallas as pl


def kernel(x, edge_index, batch, W1, b1, W2, b2):
    raise NotImplementedError("write your pallas kernel here")


# speedup vs baseline: 9.8511x; 9.8511x over previous
import functools

import jax
import jax.numpy as jnp
from jax import lax
from jax.experimental import pallas as pl
from jax.experimental.pallas import tpu as pltpu
from jax.experimental.pallas import tpu_sc as plsc

N = 100000
G = 2048
NP = 102400
GP = 2304
NC, NS = 2, 16
NW = NC * NS
EP = 1605632
SL = NP // NS
GSL = GP // NS
RB = 6400
NB = NP // RB

_mesh = plsc.VectorSubcoreMesh(
    core_axis_name="c", subcore_axis_name="s", num_cores=NC, num_subcores=NS)
_params = pltpu.CompilerParams(use_tc_tiling_on_sc=False,
                               needs_layout_passes=False)

f32 = jnp.float32
i32 = jnp.int32


@functools.partial(
    pl.kernel,
    out_type=(jax.ShapeDtypeStruct((NW * NP,), f32),
              jax.ShapeDtypeStruct((NW * GP,), f32)),
    mesh=_mesh, compiler_params=_params,
    scratch_types=[pltpu.VMEM((NP,), f32),
                   pltpu.VMEM((GP,), f32),
                   pltpu.VMEM((8, 128), i32),
                   pltpu.VMEM((25, 128), i32)])
def _sc_deg_cnt(cols_hbm, batch_hbm, zdeg_hbm,
                deg_out, cnt_out,
                dacc, cacc, cbuf, bbuf):
    c = lax.axis_index("c")
    s = lax.axis_index("s")
    wid = c * NS + s
    pltpu.sync_copy(zdeg_hbm, dacc)
    for t in range(GP // 16):
        cacc[pl.ds(t * 16, 16)] = jnp.zeros((16,), f32)

    ones = jnp.ones((16,), f32)
    iota = lax.iota(i32, 16)
    masks = [iota == k for k in range(16)]

    cpt = (EP // NW) // 128
    base = wid * cpt

    def body(g, carry):
        pltpu.sync_copy(cols_hbm.at[pl.ds(base + g * 8, 8)], cbuf)
        for j in range(8):
            for t in range(8):
                v = cbuf[j, pl.ds(t * 16, 16)]
                for k in range(16):
                    plsc.addupdate_scatter(dacc, [v], ones, mask=masks[k])
        return carry

    lax.fori_loop(0, cpt // 8, body, 0)

    pltpu.sync_copy(batch_hbm.at[wid], bbuf)
    for kk in range(25):
        for t in range(8):
            v = bbuf[kk, pl.ds(t * 16, 16)]
            for k in range(16):
                plsc.addupdate_scatter(cacc, [v], ones, mask=masks[k])

    pltpu.sync_copy(dacc, deg_out.at[pl.ds(wid * NP, NP)])
    pltpu.sync_copy(cacc, cnt_out.at[pl.ds(wid * GP, GP)])


def _edge_body(rows_hbm, cols_hbm, tab_hbm, z16_hbm, out_hbm,
               acc, rbuf, cbuf, gbuf, sem, *, cpt, gch, split_edges,
               add_core_off):
    c = lax.axis_index("c")
    s = lax.axis_index("s")
    wid = c * NS + s
    sl = pl.ds(s * SL, SL)
    pltpu.sync_copy(z16_hbm.at[sl], acc.at[sl])
    plsc.subcore_barrier()

    base = (wid if split_edges else s) * cpt
    off = c * NP

    def body(g, carry):
        gb = base + g * gch
        pltpu.sync_copy(rows_hbm.at[pl.ds(gb, gch)], rbuf)
        pltpu.sync_copy(cols_hbm.at[pl.ds(gb, gch)], cbuf)
        if add_core_off:
            for j in range(gch):
                for t in range(8):
                    rbuf[j, pl.ds(t * 16, 16)] = (
                        rbuf[j, pl.ds(t * 16, 16)] + off)
        descs = [pltpu.async_copy(tab_hbm.at[rbuf.at[j]], gbuf.at[j], sem)
                 for j in range(gch)]
        for d in descs:
            d.wait()
        for r in range(NS):
            @pl.when(s == r)
            def _():
                for j in range(gch):
                    pltpu.sync_copy(gbuf.at[j], acc.at[cbuf.at[j]], add=True)
            plsc.subcore_barrier()
        return carry

    lax.fori_loop(0, cpt // gch, body, 0)
    plsc.subcore_barrier()
    pltpu.sync_copy(acc.at[sl], out_hbm.at[pl.ds(c * NP + s * SL, SL)])


def _make_edge_pass(*, cpt, gch, split_edges, add_core_off):
    @functools.partial(
        pl.kernel,
        out_type=jax.ShapeDtypeStruct((2 * NP, 16), f32),
        mesh=_mesh, compiler_params=_params,
        scratch_types=[pltpu.VMEM_SHARED((NP, 16), f32),
                       pltpu.VMEM((gch, 128), i32),
                       pltpu.VMEM((gch, 128), i32),
                       pltpu.VMEM((gch, 128, 16), f32),
                       pltpu.SemaphoreType.DMA])
    def k(rows_hbm, cols_hbm, tab_hbm, z16_hbm, out_hbm,
          acc, rbuf, cbuf, gbuf, sem):
        _edge_body(rows_hbm, cols_hbm, tab_hbm, z16_hbm, out_hbm,
                   acc, rbuf, cbuf, gbuf, sem, cpt=cpt, gch=gch,
                   split_edges=split_edges, add_core_off=add_core_off)
    return k


_sc_edge_l1 = _make_edge_pass(cpt=(EP // NW) // 128, gch=8,
                              split_edges=True, add_core_off=False)
_sc_edge_l2 = _make_edge_pass(cpt=(EP // NS) // 128, gch=8,
                              split_edges=False, add_core_off=True)


@functools.partial(
    pl.kernel,
    out_type=jax.ShapeDtypeStruct((2 * GP, 32), f32),
    mesh=_mesh, compiler_params=_params,
    scratch_types=[pltpu.VMEM_SHARED((GP, 32), f32),
                   pltpu.VMEM((25, 128), i32),
                   pltpu.VMEM((128, 32), f32),
                   pltpu.VMEM((GSL, 32), f32)])
def _sc_pool(out2_hbm, batch_hbm, zpool_hbm,
             pool_out,
             acc, bbuf, robuf, vbuf):
    c = lax.axis_index("c")
    s = lax.axis_index("s")
    wid = c * NS + s
    gsl = pl.ds(s * GSL, GSL)
    pltpu.sync_copy(zpool_hbm.at[gsl], acc.at[gsl])
    plsc.subcore_barrier()

    pltpu.sync_copy(batch_hbm.at[wid], bbuf)
    base = wid * 3200
    for r in range(NS):
        @pl.when(s == r)
        def _():
            for k in range(25):
                pltpu.sync_copy(out2_hbm.at[pl.ds(base + k * 128, 128)],
                                robuf)
                pltpu.sync_copy(robuf, acc.at[bbuf.at[k]], add=True)
        plsc.subcore_barrier()

    pltpu.sync_copy(acc.at[gsl], vbuf)
    pltpu.sync_copy(vbuf, pool_out.at[pl.ds(c * GP + s * GSL, GSL)])


def _tc_sum_parts(src, out):
    out[...] = jnp.sum(src[...], axis=0, keepdims=True)


def _tc_prep(deg, x, dinv, gx):
    dv = lax.rsqrt(deg[...] + 1.0)
    dinv[...] = dv
    gx[...] = jnp.concatenate(
        [x[...] * dv, jnp.zeros((RB, 12), f32)], axis=1)


def _tc_layer1(a0, a1, gx, dinv, w1, b1, g2):
    dv = dinv[...]
    p1 = dv * (a0[...] + a1[...] + gx[...])
    x1 = jnp.maximum(
        jnp.dot(p1, w1[...], preferred_element_type=f32) + b1[...], 0.0)
    h = x1 * dv
    g2[0] = h[:, :16]
    g2[1] = h[:, 16:]


def _tc_layer2(a2a, a2b, g2a, g2b, dinv, w2, b2, out2):
    dv = dinv[...]
    pa = dv * (a2a[...] + g2a[0])
    pb = dv * (a2b[...] + g2b[0])
    p2 = jnp.concatenate([pa, pb], axis=1)
    out2[...] = jnp.maximum(
        jnp.dot(p2, w2[...], preferred_element_type=f32) + b2[...], 0.0)


def _tc_divide(pools, cnts, out):
    pv = pools[...]
    cv = cnts[...]
    p = pv[0:G] + pv[GP:GP + G]
    out[...] = p / jnp.maximum(cv[0:G], 1.0)


def _part_spec(width, part):
    return pl.BlockSpec((RB, width), lambda i, p=part: (i + p * NB, 0))


def _row_spec(width):
    return pl.BlockSpec((RB, width), lambda i: (i, 0))


def _full_spec(shape):
    return pl.BlockSpec(shape, lambda i: tuple(0 for _ in shape))


_reduce_deg_call = pl.pallas_call(
    _tc_sum_parts,
    grid=(NB,),
    in_specs=[pl.BlockSpec((NW, RB), lambda i: (0, i))],
    out_specs=pl.BlockSpec((1, RB), lambda i: (0, i)),
    out_shape=jax.ShapeDtypeStruct((1, NP), f32))

_reduce_cnt_call = pl.pallas_call(
    _tc_sum_parts,
    in_specs=[pl.BlockSpec((NW, GP), lambda: (0, 0))],
    out_specs=pl.BlockSpec((1, GP), lambda: (0, 0)),
    out_shape=jax.ShapeDtypeStruct((1, GP), f32))

_prep_call = pl.pallas_call(
    _tc_prep,
    grid=(NB,),
    in_specs=[_row_spec(1), _row_spec(4)],
    out_specs=[_row_spec(1), _row_spec(16)],
    out_shape=(jax.ShapeDtypeStruct((NP, 1), f32),
               jax.ShapeDtypeStruct((NP, 16), f32)))

_layer1_call = pl.pallas_call(
    _tc_layer1,
    grid=(NB,),
    in_specs=[_part_spec(16, 0), _part_spec(16, 1), _row_spec(16),
              _row_spec(1), _full_spec((16, 32)), _full_spec((1, 32))],
    out_specs=pl.BlockSpec((2, RB, 16), lambda i: (0, i, 0)),
    out_shape=jax.ShapeDtypeStruct((2, NP, 16), f32))

_layer2_call = pl.pallas_call(
    _tc_layer2,
    grid=(NB,),
    in_specs=[_part_spec(16, 0), _part_spec(16, 1),
              pl.BlockSpec((1, RB, 16), lambda i: (0, i, 0)),
              pl.BlockSpec((1, RB, 16), lambda i: (1, i, 0)),
              _row_spec(1), _full_spec((32, 32)), _full_spec((1, 32))],
    out_specs=_row_spec(32),
    out_shape=jax.ShapeDtypeStruct((NP, 32), f32))

_divide_call = pl.pallas_call(
    _tc_divide,
    in_specs=[pl.BlockSpec((2 * GP, 32), lambda: (0, 0)),
              pl.BlockSpec((GP, 1), lambda: (0, 0))],
    out_specs=pl.BlockSpec((G, 32), lambda: (0, 0)),
    out_shape=jax.ShapeDtypeStruct((G, 32), f32))


def kernel(x, edge_index, batch, W1, b1, W2, b2):
    row = edge_index[0].astype(i32)
    col = edge_index[1].astype(i32)
    e = row.shape[0]
    pad = EP - e
    rows_p = jnp.concatenate(
        [row, jnp.zeros((pad,), i32)]).reshape(EP // 128, 128)
    pad_col = N + (jnp.arange(pad, dtype=i32) % (NP - N))
    cols_p = jnp.concatenate([col, pad_col]).reshape(EP // 128, 128)
    pad_batch = G + (jnp.arange(NP - N, dtype=i32) % (GP - G))
    batch_p = jnp.concatenate(
        [batch.astype(i32), pad_batch]).reshape(NW, 25, 128)
    x_p = jnp.pad(x, ((0, NP - N), (0, 0)))
    w1p = jnp.pad(W1, ((0, 12), (0, 0)))

    zdeg = jnp.zeros((NP,), f32)
    z16 = jnp.zeros((NP, 16), f32)
    zpool = jnp.zeros((GP, 32), f32)

    deg, cnt = _sc_deg_cnt(cols_p, batch_p, zdeg)
    deg_r = _reduce_deg_call(deg.reshape(NW, NP)).reshape(NP, 1)
    cnt_r = _reduce_cnt_call(cnt.reshape(NW, GP)).reshape(GP, 1)
    dinv, gx = _prep_call(deg_r, x_p)
    a1 = _sc_edge_l1(rows_p, cols_p, gx, z16)
    g2 = _layer1_call(a1, a1, gx, dinv, w1p, b1.reshape(1, 32))
    g2f = g2.reshape(2 * NP, 16)
    a2 = _sc_edge_l2(rows_p, cols_p, g2f, z16)
    out2 = _layer2_call(a2, a2, g2, g2, dinv, W2, b2.reshape(1, 32))
    pools = _sc_pool(out2, batch_p, zpool)
    return _divide_call(pools, cnt_r)


# speedup vs baseline: 14.3458x; 1.4563x over previous
import functools

import jax
import jax.numpy as jnp
from jax import lax
from jax.experimental import pallas as pl
from jax.experimental.pallas import tpu as pltpu
from jax.experimental.pallas import tpu_sc as plsc

N = 100000
G = 2048
NP = 102400
GP = 2304
NC, NS = 2, 16
NW = NC * NS
EP = 1605632
SL = NP // NS
GSL = GP // NS
RB = 6400
NB = NP // RB

_mesh = plsc.VectorSubcoreMesh(
    core_axis_name="c", subcore_axis_name="s", num_cores=NC, num_subcores=NS)
_params = pltpu.CompilerParams(use_tc_tiling_on_sc=False,
                               needs_layout_passes=False)

f32 = jnp.float32
i32 = jnp.int32


@functools.partial(
    pl.kernel,
    out_type=(jax.ShapeDtypeStruct((NW * NP,), f32),
              jax.ShapeDtypeStruct((NW * GP,), f32)),
    mesh=_mesh, compiler_params=_params,
    scratch_types=[pltpu.VMEM((NP,), f32),
                   pltpu.VMEM((GP,), f32),
                   pltpu.VMEM((8, 128), i32),
                   pltpu.VMEM((25, 128), i32)])
def _sc_deg_cnt(cols_hbm, batch_hbm, zdeg_hbm,
                deg_out, cnt_out,
                dacc, cacc, cbuf, bbuf):
    c = lax.axis_index("c")
    s = lax.axis_index("s")
    wid = c * NS + s
    pltpu.sync_copy(zdeg_hbm, dacc)
    for t in range(GP // 16):
        cacc[pl.ds(t * 16, 16)] = jnp.zeros((16,), f32)

    ones = jnp.ones((16,), f32)
    iota = lax.iota(i32, 16)
    masks = [iota == k for k in range(16)]

    cpt = (EP // NW) // 128
    base = wid * cpt

    def body(g, carry):
        pltpu.sync_copy(cols_hbm.at[pl.ds(base + g * 8, 8)], cbuf)
        for j in range(8):
            for t in range(8):
                v = cbuf[j, pl.ds(t * 16, 16)]
                for k in range(16):
                    plsc.addupdate_scatter(dacc, [v], ones, mask=masks[k])
        return carry

    lax.fori_loop(0, cpt // 8, body, 0)

    pltpu.sync_copy(batch_hbm.at[wid], bbuf)
    for kk in range(25):
        for t in range(8):
            v = bbuf[kk, pl.ds(t * 16, 16)]
            for k in range(16):
                plsc.addupdate_scatter(cacc, [v], ones, mask=masks[k])

    pltpu.sync_copy(dacc, deg_out.at[pl.ds(wid * NP, NP)])
    pltpu.sync_copy(cacc, cnt_out.at[pl.ds(wid * GP, GP)])


def _edge_body(rows_hbm, cols_hbm, tab_hbm, z16_hbm, out_hbm,
               acc, rbuf, cbuf, gbuf, sem, *, cpt, gch, split_edges,
               add_core_off):
    c = lax.axis_index("c")
    s = lax.axis_index("s")
    wid = c * NS + s
    sl = pl.ds(s * SL, SL)
    pltpu.sync_copy(z16_hbm.at[sl], acc.at[sl])
    plsc.subcore_barrier()

    base = (wid if split_edges else s) * cpt * 128
    off = c * NP

    def body(g, carry):
        gb = base + g * gch * 128
        pltpu.sync_copy(rows_hbm.at[pl.ds(gb, gch * 128)], rbuf)
        pltpu.sync_copy(cols_hbm.at[pl.ds(gb, gch * 128)], cbuf)
        if add_core_off:
            for t in range(gch * 8):
                rbuf[pl.ds(t * 16, 16)] = rbuf[pl.ds(t * 16, 16)] + off
        pltpu.async_copy(tab_hbm.at[rbuf], gbuf, sem).wait()
        for r in range(NS):
            @pl.when(s == r)
            def _():
                pltpu.sync_copy(gbuf, acc.at[cbuf], add=True)
            plsc.subcore_barrier()
        return carry

    lax.fori_loop(0, cpt // gch, body, 0)
    plsc.subcore_barrier()
    pltpu.sync_copy(acc.at[sl], out_hbm.at[pl.ds(c * NP + s * SL, SL)])


def _make_edge_pass(*, cpt, gch, split_edges, add_core_off):
    @functools.partial(
        pl.kernel,
        out_type=jax.ShapeDtypeStruct((2 * NP, 16), f32),
        mesh=_mesh, compiler_params=_params,
        scratch_types=[pltpu.VMEM_SHARED((NP, 16), f32),
                       pltpu.VMEM((gch * 128,), i32),
                       pltpu.VMEM((gch * 128,), i32),
                       pltpu.VMEM((gch * 128, 16), f32),
                       pltpu.SemaphoreType.DMA])
    def k(rows_hbm, cols_hbm, tab_hbm, z16_hbm, out_hbm,
          acc, rbuf, cbuf, gbuf, sem):
        _edge_body(rows_hbm, cols_hbm, tab_hbm, z16_hbm, out_hbm,
                   acc, rbuf, cbuf, gbuf, sem, cpt=cpt, gch=gch,
                   split_edges=split_edges, add_core_off=add_core_off)
    return k


_sc_edge_l1 = _make_edge_pass(cpt=(EP // NW) // 128, gch=8,
                              split_edges=True, add_core_off=False)
_sc_edge_l2 = _make_edge_pass(cpt=(EP // NS) // 128, gch=8,
                              split_edges=False, add_core_off=True)


@functools.partial(
    pl.kernel,
    out_type=jax.ShapeDtypeStruct((2 * GP, 32), f32),
    mesh=_mesh, compiler_params=_params,
    scratch_types=[pltpu.VMEM_SHARED((GP, 32), f32),
                   pltpu.VMEM((5 * 128,), i32),
                   pltpu.VMEM((5 * 128, 32), f32),
                   pltpu.VMEM((GSL, 32), f32),
                   pltpu.SemaphoreType.DMA])
def _sc_pool(out2_hbm, batchf_hbm, zpool_hbm,
             pool_out,
             acc, bbuf, robuf, vbuf, sem):
    c = lax.axis_index("c")
    s = lax.axis_index("s")
    wid = c * NS + s
    gsl = pl.ds(s * GSL, GSL)
    pltpu.sync_copy(zpool_hbm.at[gsl], acc.at[gsl])
    plsc.subcore_barrier()

    base = wid * 3200
    for g in range(5):
        pltpu.sync_copy(
            batchf_hbm.at[pl.ds(base + g * 640, 640)], bbuf)
        pltpu.async_copy(
            out2_hbm.at[pl.ds(base + g * 640, 640)], robuf, sem).wait()
        for r in range(NS):
            @pl.when(s == r)
            def _():
                pltpu.sync_copy(robuf, acc.at[bbuf], add=True)
            plsc.subcore_barrier()

    pltpu.sync_copy(acc.at[gsl], vbuf)
    pltpu.sync_copy(vbuf, pool_out.at[pl.ds(c * GP + s * GSL, GSL)])


def _tc_sum_parts(src, out):
    out[...] = jnp.sum(src[...], axis=0, keepdims=True)


def _tc_prep(deg, x, dinv, gx):
    dv = lax.rsqrt(deg[...] + 1.0)
    dinv[...] = dv
    gx[...] = jnp.concatenate(
        [x[...] * dv, jnp.zeros((RB, 12), f32)], axis=1)


def _tc_layer1(a0, a1, gx, dinv, w1, b1, g2):
    dv = dinv[...]
    p1 = dv * (a0[...] + a1[...] + gx[...])
    x1 = jnp.maximum(
        jnp.dot(p1, w1[...], preferred_element_type=f32) + b1[...], 0.0)
    h = x1 * dv
    g2[0] = h[:, :16]
    g2[1] = h[:, 16:]


def _tc_layer2(a2a, a2b, g2a, g2b, dinv, w2, b2, out2):
    dv = dinv[...]
    pa = dv * (a2a[...] + g2a[0])
    pb = dv * (a2b[...] + g2b[0])
    p2 = jnp.concatenate([pa, pb], axis=1)
    out2[...] = jnp.maximum(
        jnp.dot(p2, w2[...], preferred_element_type=f32) + b2[...], 0.0)


def _tc_divide(pools, cnts, out):
    pv = pools[...]
    cv = cnts[...]
    p = pv[0:G] + pv[GP:GP + G]
    out[...] = p / jnp.maximum(cv[0:G], 1.0)


def _part_spec(width, part):
    return pl.BlockSpec((RB, width), lambda i, p=part: (i + p * NB, 0))


def _row_spec(width):
    return pl.BlockSpec((RB, width), lambda i: (i, 0))


def _full_spec(shape):
    return pl.BlockSpec(shape, lambda i: tuple(0 for _ in shape))


_reduce_deg_call = pl.pallas_call(
    _tc_sum_parts,
    grid=(NB,),
    in_specs=[pl.BlockSpec((NW, RB), lambda i: (0, i))],
    out_specs=pl.BlockSpec((1, RB), lambda i: (0, i)),
    out_shape=jax.ShapeDtypeStruct((1, NP), f32))

_reduce_cnt_call = pl.pallas_call(
    _tc_sum_parts,
    in_specs=[pl.BlockSpec((NW, GP), lambda: (0, 0))],
    out_specs=pl.BlockSpec((1, GP), lambda: (0, 0)),
    out_shape=jax.ShapeDtypeStruct((1, GP), f32))

_prep_call = pl.pallas_call(
    _tc_prep,
    grid=(NB,),
    in_specs=[_row_spec(1), _row_spec(4)],
    out_specs=[_row_spec(1), _row_spec(16)],
    out_shape=(jax.ShapeDtypeStruct((NP, 1), f32),
               jax.ShapeDtypeStruct((NP, 16), f32)))

_layer1_call = pl.pallas_call(
    _tc_layer1,
    grid=(NB,),
    in_specs=[_part_spec(16, 0), _part_spec(16, 1), _row_spec(16),
              _row_spec(1), _full_spec((16, 32)), _full_spec((1, 32))],
    out_specs=pl.BlockSpec((2, RB, 16), lambda i: (0, i, 0)),
    out_shape=jax.ShapeDtypeStruct((2, NP, 16), f32))

_layer2_call = pl.pallas_call(
    _tc_layer2,
    grid=(NB,),
    in_specs=[_part_spec(16, 0), _part_spec(16, 1),
              pl.BlockSpec((1, RB, 16), lambda i: (0, i, 0)),
              pl.BlockSpec((1, RB, 16), lambda i: (1, i, 0)),
              _row_spec(1), _full_spec((32, 32)), _full_spec((1, 32))],
    out_specs=_row_spec(32),
    out_shape=jax.ShapeDtypeStruct((NP, 32), f32))

_divide_call = pl.pallas_call(
    _tc_divide,
    in_specs=[pl.BlockSpec((2 * GP, 32), lambda: (0, 0)),
              pl.BlockSpec((GP, 1), lambda: (0, 0))],
    out_specs=pl.BlockSpec((G, 32), lambda: (0, 0)),
    out_shape=jax.ShapeDtypeStruct((G, 32), f32))


def kernel(x, edge_index, batch, W1, b1, W2, b2):
    row = edge_index[0].astype(i32)
    col = edge_index[1].astype(i32)
    e = row.shape[0]
    pad = EP - e
    rows_p = jnp.concatenate(
        [row, jnp.zeros((pad,), i32)]).reshape(EP // 128, 128)
    pad_col = N + (jnp.arange(pad, dtype=i32) % (NP - N))
    cols_p = jnp.concatenate([col, pad_col]).reshape(EP // 128, 128)
    pad_batch = G + (jnp.arange(NP - N, dtype=i32) % (GP - G))
    batch_p = jnp.concatenate(
        [batch.astype(i32), pad_batch]).reshape(NW, 25, 128)
    x_p = jnp.pad(x, ((0, NP - N), (0, 0)))
    w1p = jnp.pad(W1, ((0, 12), (0, 0)))

    zdeg = jnp.zeros((NP,), f32)
    z16 = jnp.zeros((NP, 16), f32)
    zpool = jnp.zeros((GP, 32), f32)

    deg, cnt = _sc_deg_cnt(cols_p, batch_p, zdeg)
    deg_r = _reduce_deg_call(deg.reshape(NW, NP)).reshape(NP, 1)
    cnt_r = _reduce_cnt_call(cnt.reshape(NW, GP)).reshape(GP, 1)
    dinv, gx = _prep_call(deg_r, x_p)
    rows_f = rows_p.reshape(EP)
    cols_f = cols_p.reshape(EP)
    a1 = _sc_edge_l1(rows_f, cols_f, gx, z16)
    g2 = _layer1_call(a1, a1, gx, dinv, w1p, b1.reshape(1, 32))
    g2f = g2.reshape(2 * NP, 16)
    a2 = _sc_edge_l2(rows_f, cols_f, g2f, z16)
    out2 = _layer2_call(a2, a2, g2, g2, dinv, W2, b2.reshape(1, 32))
    batch_f = batch_p.reshape(NP)
    pools = _sc_pool(out2, batch_f, zpool)
    return _divide_call(pools, cnt_r)


# speedup vs baseline: 15.4298x; 1.0756x over previous
import functools

import jax
import jax.numpy as jnp
from jax import lax
from jax.experimental import pallas as pl
from jax.experimental.pallas import tpu as pltpu
from jax.experimental.pallas import tpu_sc as plsc

N = 100000
G = 2048
NP = 100352
GP = 2304
NC, NS = 2, 16
NW = NC * NS
EP = 1605632
SL = NP // NS
GSL = GP // NS
RB = 3136
NB = NP // RB
PT = NP // NW

_mesh = plsc.VectorSubcoreMesh(
    core_axis_name="c", subcore_axis_name="s", num_cores=NC, num_subcores=NS)
_params = pltpu.CompilerParams(use_tc_tiling_on_sc=False,
                               needs_layout_passes=False)

f32 = jnp.float32
i32 = jnp.int32


@functools.partial(
    pl.kernel,
    out_type=(jax.ShapeDtypeStruct((NW * NP,), f32),
              jax.ShapeDtypeStruct((NW * GP,), f32)),
    mesh=_mesh, compiler_params=_params,
    scratch_types=[pltpu.VMEM((NP,), f32),
                   pltpu.VMEM((GP,), f32),
                   pltpu.VMEM((8, 128), i32),
                   pltpu.VMEM((PT,), i32)])
def _sc_deg_cnt(cols_hbm, batch_hbm, zdeg_hbm,
                deg_out, cnt_out,
                dacc, cacc, cbuf, bbuf):
    c = lax.axis_index("c")
    s = lax.axis_index("s")
    wid = c * NS + s
    pltpu.sync_copy(zdeg_hbm, dacc)
    for t in range(GP // 16):
        cacc[pl.ds(t * 16, 16)] = jnp.zeros((16,), f32)

    ones = jnp.ones((16,), f32)
    iota = lax.iota(i32, 16)
    masks = [iota == k for k in range(16)]

    cpt = (EP // NW) // 128
    base = wid * cpt

    def body(g, carry):
        pltpu.sync_copy(cols_hbm.at[pl.ds(base + g * 8, 8)], cbuf)
        for j in range(8):
            for t in range(8):
                v = cbuf[j, pl.ds(t * 16, 16)]
                for k in range(16):
                    plsc.addupdate_scatter(dacc, [v], ones, mask=masks[k])
        return carry

    lax.fori_loop(0, cpt // 8, body, 0)

    pltpu.sync_copy(batch_hbm.at[pl.ds(wid * PT, PT)], bbuf)

    def bbody(t, carry):
        v = bbuf[pl.ds(t * 16, 16)]
        for k in range(16):
            plsc.addupdate_scatter(cacc, [v], ones, mask=masks[k])
        return carry

    lax.fori_loop(0, PT // 16, bbody, 0)

    pltpu.sync_copy(dacc, deg_out.at[pl.ds(wid * NP, NP)])
    pltpu.sync_copy(cacc, cnt_out.at[pl.ds(wid * GP, GP)])


def _edge_body(rows_hbm, cols_hbm, tab_hbm, z16_hbm, out_hbm,
               acc, ibuf, gbuf, sem, *, cpt, gch, split_edges,
               add_core_off):
    c = lax.axis_index("c")
    s = lax.axis_index("s")
    wid = c * NS + s
    sl = pl.ds(s * SL, SL)
    pltpu.sync_copy(z16_hbm.at[sl], acc.at[sl])
    plsc.subcore_barrier()

    base = (wid if split_edges else s) * cpt * 128
    off = c * NP

    def body(g, carry):
        gb = base + g * gch * 128
        pltpu.sync_copy(rows_hbm.at[pl.ds(gb, gch * 128)], ibuf)
        if add_core_off:
            for t in range(gch * 8):
                ibuf[pl.ds(t * 16, 16)] = ibuf[pl.ds(t * 16, 16)] + off
        pltpu.async_copy(tab_hbm.at[ibuf], gbuf, sem).wait()
        pltpu.sync_copy(cols_hbm.at[pl.ds(gb, gch * 128)], ibuf)
        for r in range(NS):
            @pl.when(s == r)
            def _():
                pltpu.sync_copy(gbuf, acc.at[ibuf], add=True)
            plsc.subcore_barrier()
        return carry

    lax.fori_loop(0, cpt // gch, body, 0)
    plsc.subcore_barrier()
    pltpu.sync_copy(acc.at[sl], out_hbm.at[pl.ds(c * NP + s * SL, SL)])


def _make_edge_pass(*, cpt, gch, split_edges, add_core_off):
    @functools.partial(
        pl.kernel,
        out_type=jax.ShapeDtypeStruct((2 * NP, 16), f32),
        mesh=_mesh, compiler_params=_params,
        scratch_types=[pltpu.VMEM_SHARED((NP, 16), f32),
                       pltpu.VMEM((gch * 128,), i32),
                       pltpu.VMEM((gch * 128, 16), f32),
                       pltpu.SemaphoreType.DMA])
    def k(rows_hbm, cols_hbm, tab_hbm, z16_hbm, out_hbm,
          acc, ibuf, gbuf, sem):
        _edge_body(rows_hbm, cols_hbm, tab_hbm, z16_hbm, out_hbm,
                   acc, ibuf, gbuf, sem, cpt=cpt, gch=gch,
                   split_edges=split_edges, add_core_off=add_core_off)
    return k


_sc_edge_l1 = _make_edge_pass(cpt=(EP // NW) // 128, gch=14,
                              split_edges=True, add_core_off=False)
_sc_edge_l2 = _make_edge_pass(cpt=(EP // NS) // 128, gch=14,
                              split_edges=False, add_core_off=True)


@functools.partial(
    pl.kernel,
    out_type=jax.ShapeDtypeStruct((2 * GP, 32), f32),
    mesh=_mesh, compiler_params=_params,
    scratch_types=[pltpu.VMEM_SHARED((GP, 32), f32),
                   pltpu.VMEM((448,), i32),
                   pltpu.VMEM((448, 32), f32),
                   pltpu.VMEM((GSL, 32), f32),
                   pltpu.SemaphoreType.DMA])
def _sc_pool(out2_hbm, batchf_hbm, zpool_hbm,
             pool_out,
             acc, bbuf, robuf, vbuf, sem):
    c = lax.axis_index("c")
    s = lax.axis_index("s")
    wid = c * NS + s
    gsl = pl.ds(s * GSL, GSL)
    pltpu.sync_copy(zpool_hbm.at[gsl], acc.at[gsl])
    plsc.subcore_barrier()

    base = wid * PT
    for g in range(7):
        pltpu.sync_copy(
            batchf_hbm.at[pl.ds(base + g * 448, 448)], bbuf)
        pltpu.async_copy(
            out2_hbm.at[pl.ds(base + g * 448, 448)], robuf, sem).wait()
        for r in range(NS):
            @pl.when(s == r)
            def _():
                pltpu.sync_copy(robuf, acc.at[bbuf], add=True)
            plsc.subcore_barrier()

    pltpu.sync_copy(acc.at[gsl], vbuf)
    pltpu.sync_copy(vbuf, pool_out.at[pl.ds(c * GP + s * GSL, GSL)])


def _tc_sum_parts(src, out):
    out[...] = jnp.sum(src[...], axis=0, keepdims=True)


def _tc_prep(deg, x, dinv, gx):
    dv = lax.rsqrt(deg[...] + 1.0)
    dinv[...] = dv
    gx[...] = jnp.concatenate(
        [x[...] * dv, jnp.zeros((RB, 12), f32)], axis=1)


def _tc_layer1(a0, a1, gx, dinv, w1, b1, g2):
    dv = dinv[...]
    p1 = dv * (a0[...] + a1[...] + gx[...])
    x1 = jnp.maximum(
        jnp.dot(p1, w1[...], preferred_element_type=f32) + b1[...], 0.0)
    h = x1 * dv
    g2[0] = h[:, :16]
    g2[1] = h[:, 16:]


def _tc_layer2(a2a, a2b, g2a, g2b, dinv, w2, b2, out2):
    dv = dinv[...]
    pa = dv * (a2a[...] + g2a[0])
    pb = dv * (a2b[...] + g2b[0])
    p2 = jnp.concatenate([pa, pb], axis=1)
    out2[...] = jnp.maximum(
        jnp.dot(p2, w2[...], preferred_element_type=f32) + b2[...], 0.0)


def _tc_divide(pools, cnts, out):
    pv = pools[...]
    cv = cnts[...]
    p = pv[0:G] + pv[GP:GP + G]
    out[...] = p / jnp.maximum(cv[0:G], 1.0)


def _part_spec(width, part):
    return pl.BlockSpec((RB, width), lambda i, p=part: (i + p * NB, 0))


def _row_spec(width):
    return pl.BlockSpec((RB, width), lambda i: (i, 0))


def _full_spec(shape):
    return pl.BlockSpec(shape, lambda i: tuple(0 for _ in shape))


_reduce_deg_call = pl.pallas_call(
    _tc_sum_parts,
    grid=(NP // 6272,),
    in_specs=[pl.BlockSpec((NW, 6272), lambda i: (0, i))],
    out_specs=pl.BlockSpec((1, 6272), lambda i: (0, i)),
    out_shape=jax.ShapeDtypeStruct((1, NP), f32))

_reduce_cnt_call = pl.pallas_call(
    _tc_sum_parts,
    in_specs=[pl.BlockSpec((NW, GP), lambda: (0, 0))],
    out_specs=pl.BlockSpec((1, GP), lambda: (0, 0)),
    out_shape=jax.ShapeDtypeStruct((1, GP), f32))

_prep_call = pl.pallas_call(
    _tc_prep,
    grid=(NB,),
    in_specs=[_row_spec(1), _row_spec(4)],
    out_specs=[_row_spec(1), _row_spec(16)],
    out_shape=(jax.ShapeDtypeStruct((NP, 1), f32),
               jax.ShapeDtypeStruct((NP, 16), f32)))

_layer1_call = pl.pallas_call(
    _tc_layer1,
    grid=(NB,),
    in_specs=[_part_spec(16, 0), _part_spec(16, 1), _row_spec(16),
              _row_spec(1), _full_spec((16, 32)), _full_spec((1, 32))],
    out_specs=pl.BlockSpec((2, RB, 16), lambda i: (0, i, 0)),
    out_shape=jax.ShapeDtypeStruct((2, NP, 16), f32))

_layer2_call = pl.pallas_call(
    _tc_layer2,
    grid=(NB,),
    in_specs=[_part_spec(16, 0), _part_spec(16, 1),
              pl.BlockSpec((1, RB, 16), lambda i: (0, i, 0)),
              pl.BlockSpec((1, RB, 16), lambda i: (1, i, 0)),
              _row_spec(1), _full_spec((32, 32)), _full_spec((1, 32))],
    out_specs=_row_spec(32),
    out_shape=jax.ShapeDtypeStruct((NP, 32), f32))

_divide_call = pl.pallas_call(
    _tc_divide,
    in_specs=[pl.BlockSpec((2 * GP, 32), lambda: (0, 0)),
              pl.BlockSpec((GP, 1), lambda: (0, 0))],
    out_specs=pl.BlockSpec((G, 32), lambda: (0, 0)),
    out_shape=jax.ShapeDtypeStruct((G, 32), f32))


def kernel(x, edge_index, batch, W1, b1, W2, b2):
    row = edge_index[0].astype(i32)
    col = edge_index[1].astype(i32)
    e = row.shape[0]
    pad = EP - e
    rows_p = jnp.concatenate(
        [row, jnp.zeros((pad,), i32)]).reshape(EP // 128, 128)
    pad_col = N + (jnp.arange(pad, dtype=i32) % (NP - N))
    cols_p = jnp.concatenate([col, pad_col]).reshape(EP // 128, 128)
    pad_batch = G + (jnp.arange(NP - N, dtype=i32) % (GP - G))
    batch_p = jnp.concatenate([batch.astype(i32), pad_batch])
    x_p = jnp.pad(x, ((0, NP - N), (0, 0)))
    w1p = jnp.pad(W1, ((0, 12), (0, 0)))

    zdeg = jnp.zeros((NP,), f32)
    z16 = jnp.zeros((NP, 16), f32)
    zpool = jnp.zeros((GP, 32), f32)

    deg, cnt = _sc_deg_cnt(cols_p, batch_p, zdeg)
    deg_r = _reduce_deg_call(deg.reshape(NW, NP)).reshape(NP, 1)
    cnt_r = _reduce_cnt_call(cnt.reshape(NW, GP)).reshape(GP, 1)
    dinv, gx = _prep_call(deg_r, x_p)
    rows_f = rows_p.reshape(EP)
    cols_f = cols_p.reshape(EP)
    a1 = _sc_edge_l1(rows_f, cols_f, gx, z16)
    g2 = _layer1_call(a1, a1, gx, dinv, w1p, b1.reshape(1, 32))
    g2f = g2.reshape(2 * NP, 16)
    a2 = _sc_edge_l2(rows_f, cols_f, g2f, z16)
    out2 = _layer2_call(a2, a2, g2, g2, dinv, W2, b2.reshape(1, 32))
    pools = _sc_pool(out2, batch_p, zpool)
    return _divide_call(pools, cnt_r)


# speedup vs baseline: 16.0909x; 1.0428x over previous
import functools

import jax
import jax.numpy as jnp
from jax import lax
from jax.experimental import pallas as pl
from jax.experimental.pallas import tpu as pltpu
from jax.experimental.pallas import tpu_sc as plsc

N = 100000
G = 2048
NP = 100352
GP = 2304
NC, NS = 2, 16
NW = NC * NS
EP = 1605632
SL = NP // NS
GSL = GP // NS
RB = 3136
NB = NP // RB
PT = NP // NW

_mesh = plsc.VectorSubcoreMesh(
    core_axis_name="c", subcore_axis_name="s", num_cores=NC, num_subcores=NS)
_params = pltpu.CompilerParams(use_tc_tiling_on_sc=False,
                               needs_layout_passes=False)

f32 = jnp.float32
i32 = jnp.int32


@functools.partial(
    pl.kernel,
    out_type=(jax.ShapeDtypeStruct((NW * NP,), f32),
              jax.ShapeDtypeStruct((NW * GP,), f32)),
    mesh=_mesh, compiler_params=_params,
    scratch_types=[pltpu.VMEM((NP,), f32),
                   pltpu.VMEM((GP,), f32),
                   pltpu.VMEM((8, 128), i32),
                   pltpu.VMEM((PT,), i32)])
def _sc_deg_cnt(cols_hbm, batch_hbm, zdeg_hbm,
                deg_out, cnt_out,
                dacc, cacc, cbuf, bbuf):
    c = lax.axis_index("c")
    s = lax.axis_index("s")
    wid = c * NS + s
    pltpu.sync_copy(zdeg_hbm, dacc)
    for t in range(GP // 16):
        cacc[pl.ds(t * 16, 16)] = jnp.zeros((16,), f32)

    ones = jnp.ones((16,), f32)
    iota = lax.iota(i32, 16)
    masks = [iota == k for k in range(16)]

    cpt = (EP // NW) // 128
    base = wid * cpt

    def body(g, carry):
        pltpu.sync_copy(cols_hbm.at[pl.ds(base + g * 8, 8)], cbuf)
        for j in range(8):
            for t in range(8):
                v = cbuf[j, pl.ds(t * 16, 16)]
                for k in range(16):
                    plsc.addupdate_scatter(dacc, [v], ones, mask=masks[k])
        return carry

    lax.fori_loop(0, cpt // 8, body, 0)

    pltpu.sync_copy(batch_hbm.at[pl.ds(wid * PT, PT)], bbuf)

    def bbody(t, carry):
        v = bbuf[pl.ds(t * 16, 16)]
        for k in range(16):
            plsc.addupdate_scatter(cacc, [v], ones, mask=masks[k])
        return carry

    lax.fori_loop(0, PT // 16, bbody, 0)

    pltpu.sync_copy(dacc, deg_out.at[pl.ds(wid * NP, NP)])
    pltpu.sync_copy(cacc, cnt_out.at[pl.ds(wid * GP, GP)])


def _edge_body(rows_hbm, cols_hbm, tab_hbm, z16_hbm, out_hbm,
               acc, ibuf, gbuf, sem, *, cpt, gch, split_edges,
               add_core_off):
    c = lax.axis_index("c")
    s = lax.axis_index("s")
    wid = c * NS + s
    sl = pl.ds(s * SL, SL)
    pltpu.sync_copy(z16_hbm.at[sl], acc.at[sl])
    plsc.subcore_barrier()

    base = (wid if split_edges else s) * cpt * 128
    off = c * NP

    def body(g, carry):
        gb = base + g * gch * 128
        pltpu.sync_copy(rows_hbm.at[pl.ds(gb, gch * 128)], ibuf)
        if add_core_off:
            for t in range(gch * 8):
                ibuf[pl.ds(t * 16, 16)] = ibuf[pl.ds(t * 16, 16)] + off
        pltpu.async_copy(tab_hbm.at[ibuf], gbuf, sem).wait()
        pltpu.sync_copy(cols_hbm.at[pl.ds(gb, gch * 128)], ibuf)
        for r in range(NS):
            @pl.when(s == r)
            def _():
                pltpu.sync_copy(gbuf, acc.at[ibuf], add=True)
            plsc.subcore_barrier()
        return carry

    lax.fori_loop(0, cpt // gch, body, 0)
    plsc.subcore_barrier()
    pltpu.sync_copy(acc.at[sl], out_hbm.at[pl.ds(c * NP + s * SL, SL)])


def _make_edge_pass(*, cpt, gch, split_edges, add_core_off):
    @functools.partial(
        pl.kernel,
        out_type=jax.ShapeDtypeStruct((2 * NP, 16), f32),
        mesh=_mesh, compiler_params=_params,
        scratch_types=[pltpu.VMEM_SHARED((NP, 16), f32),
                       pltpu.VMEM((gch * 128,), i32),
                       pltpu.VMEM((gch * 128, 16), f32),
                       pltpu.SemaphoreType.DMA])
    def k(rows_hbm, cols_hbm, tab_hbm, z16_hbm, out_hbm,
          acc, ibuf, gbuf, sem):
        _edge_body(rows_hbm, cols_hbm, tab_hbm, z16_hbm, out_hbm,
                   acc, ibuf, gbuf, sem, cpt=cpt, gch=gch,
                   split_edges=split_edges, add_core_off=add_core_off)
    return k


_sc_edge_l2 = _make_edge_pass(cpt=(EP // NS) // 128, gch=14,
                              split_edges=False, add_core_off=True)

NP4 = NP // 4


@functools.partial(
    pl.kernel,
    out_type=jax.ShapeDtypeStruct((2 * NP, 16), f32),
    mesh=_mesh, compiler_params=_params,
    scratch_types=[pltpu.VMEM_SHARED((NP, 16), f32),
                   pltpu.VMEM((1024,), i32),
                   pltpu.VMEM((1024,), i32),
                   pltpu.VMEM((1024, 16), f32),
                   pltpu.SemaphoreType.DMA])
def _sc_edge_l1(rows_hbm, cols_hbm, tab4_hbm, z16_hbm, out_hbm,
                acc, rbuf, cbuf, gbuf, sem):
    c = lax.axis_index("c")
    s = lax.axis_index("s")
    wid = c * NS + s
    sl = pl.ds(s * SL, SL)
    pltpu.sync_copy(z16_hbm.at[sl], acc.at[sl])
    plsc.subcore_barrier()

    cpt = (EP // NW) // 128
    base = wid * cpt * 128
    qbase = (s // 4) * NP4

    def body(g, carry):
        gb = base + g * 1024
        pltpu.sync_copy(rows_hbm.at[pl.ds(gb, 1024)], rbuf)
        pltpu.sync_copy(cols_hbm.at[pl.ds(gb, 1024)], cbuf)
        for t in range(64):
            dst = pl.ds(t * 16, 16)
            rv = rbuf[dst]
            cv = cbuf[dst]
            rbuf[dst] = rv * 4 + jnp.bitwise_and(cv, 3)
            cbuf[dst] = lax.shift_right_logical(cv, 2) + qbase
        pltpu.async_copy(tab4_hbm.at[rbuf], gbuf, sem).wait()
        for r in range(4):
            @pl.when(s % 4 == r)
            def _():
                pltpu.sync_copy(gbuf, acc.at[cbuf], add=True)
            plsc.subcore_barrier()
        return carry

    lax.fori_loop(0, cpt // 8, body, 0)
    plsc.subcore_barrier()
    pltpu.sync_copy(acc.at[sl], out_hbm.at[pl.ds(c * NP + s * SL, SL)])


@functools.partial(
    pl.kernel,
    out_type=jax.ShapeDtypeStruct((2 * GP, 32), f32),
    mesh=_mesh, compiler_params=_params,
    scratch_types=[pltpu.VMEM_SHARED((GP, 32), f32),
                   pltpu.VMEM((448,), i32),
                   pltpu.VMEM((448, 32), f32),
                   pltpu.VMEM((GSL, 32), f32),
                   pltpu.SemaphoreType.DMA])
def _sc_pool(out2_hbm, batchf_hbm, zpool_hbm,
             pool_out,
             acc, bbuf, robuf, vbuf, sem):
    c = lax.axis_index("c")
    s = lax.axis_index("s")
    wid = c * NS + s
    gsl = pl.ds(s * GSL, GSL)
    pltpu.sync_copy(zpool_hbm.at[gsl], acc.at[gsl])
    plsc.subcore_barrier()

    base = wid * PT
    for g in range(7):
        pltpu.sync_copy(
            batchf_hbm.at[pl.ds(base + g * 448, 448)], bbuf)
        pltpu.async_copy(
            out2_hbm.at[pl.ds(base + g * 448, 448)], robuf, sem).wait()
        for r in range(NS):
            @pl.when(s == r)
            def _():
                pltpu.sync_copy(robuf, acc.at[bbuf], add=True)
            plsc.subcore_barrier()

    pltpu.sync_copy(acc.at[gsl], vbuf)
    pltpu.sync_copy(vbuf, pool_out.at[pl.ds(c * GP + s * GSL, GSL)])


def _tc_sum_parts(src, out):
    out[...] = jnp.sum(src[...], axis=0, keepdims=True)


def _tc_prep(deg, x, dinv, gx, tab4):
    dv = lax.rsqrt(deg[...] + 1.0)
    dinv[...] = dv
    g = x[...] * dv
    gx[...] = g
    z = jnp.zeros((RB, 4), f32)
    tab4[:, 0, :] = jnp.concatenate([g, z, z, z], axis=1)
    tab4[:, 1, :] = jnp.concatenate([z, g, z, z], axis=1)
    tab4[:, 2, :] = jnp.concatenate([z, z, g, z], axis=1)
    tab4[:, 3, :] = jnp.concatenate([z, z, z, g], axis=1)


def _tc_l1sum(*refs):
    parts = refs[:8]
    gx4 = refs[8]
    u4 = refs[9]
    acc = gx4[...]
    for p in parts:
        acc = acc + p[...]
    u4[...] = acc


def _tc_layer1(u, dinv, w1, b1, g2):
    dv = dinv[...]
    p1 = dv * u[...]
    x1 = jnp.maximum(
        jnp.dot(p1, w1[...], preferred_element_type=f32) + b1[...], 0.0)
    h = x1 * dv
    g2[0] = h[:, :16]
    g2[1] = h[:, 16:]


def _tc_layer2(a2a, a2b, g2a, g2b, dinv, w2, b2, out2):
    dv = dinv[...]
    pa = dv * (a2a[...] + g2a[0])
    pb = dv * (a2b[...] + g2b[0])
    p2 = jnp.concatenate([pa, pb], axis=1)
    out2[...] = jnp.maximum(
        jnp.dot(p2, w2[...], preferred_element_type=f32) + b2[...], 0.0)


def _tc_divide(pools, cnts, out):
    pv = pools[...]
    cv = cnts[...]
    p = pv[0:G] + pv[GP:GP + G]
    out[...] = p / jnp.maximum(cv[0:G], 1.0)


def _part_spec(width, part):
    return pl.BlockSpec((RB, width), lambda i, p=part: (i + p * NB, 0))


def _row_spec(width):
    return pl.BlockSpec((RB, width), lambda i: (i, 0))


def _full_spec(shape):
    return pl.BlockSpec(shape, lambda i: tuple(0 for _ in shape))


_reduce_deg_call = pl.pallas_call(
    _tc_sum_parts,
    grid=(NP // 6272,),
    in_specs=[pl.BlockSpec((NW, 6272), lambda i: (0, i))],
    out_specs=pl.BlockSpec((1, 6272), lambda i: (0, i)),
    out_shape=jax.ShapeDtypeStruct((1, NP), f32))

_reduce_cnt_call = pl.pallas_call(
    _tc_sum_parts,
    in_specs=[pl.BlockSpec((NW, GP), lambda: (0, 0))],
    out_specs=pl.BlockSpec((1, GP), lambda: (0, 0)),
    out_shape=jax.ShapeDtypeStruct((1, GP), f32))

_prep_call = pl.pallas_call(
    _tc_prep,
    grid=(NB,),
    in_specs=[_row_spec(1), _row_spec(4)],
    out_specs=[_row_spec(1), _row_spec(4),
               pl.BlockSpec((RB, 4, 16), lambda i: (i, 0, 0))],
    out_shape=(jax.ShapeDtypeStruct((NP, 1), f32),
               jax.ShapeDtypeStruct((NP, 4), f32),
               jax.ShapeDtypeStruct((NP, 4, 16), f32)))

_l1sum_call = pl.pallas_call(
    _tc_l1sum,
    grid=(NP4 // RB,),
    in_specs=([pl.BlockSpec((RB, 16), lambda i, q=q: (i + q * (NP4 // RB), 0))
               for q in range(4)] +
              [pl.BlockSpec((RB, 16),
                            lambda i, q=q: (i + NB + q * (NP4 // RB), 0))
               for q in range(4)] +
              [pl.BlockSpec((RB, 16), lambda i: (i, 0))]),
    out_specs=pl.BlockSpec((RB, 16), lambda i: (i, 0)),
    out_shape=jax.ShapeDtypeStruct((NP4, 16), f32))

_layer1_call = pl.pallas_call(
    _tc_layer1,
    grid=(NB,),
    in_specs=[_row_spec(4),
              _row_spec(1), _full_spec((4, 32)), _full_spec((1, 32))],
    out_specs=pl.BlockSpec((2, RB, 16), lambda i: (0, i, 0)),
    out_shape=jax.ShapeDtypeStruct((2, NP, 16), f32))

_layer2_call = pl.pallas_call(
    _tc_layer2,
    grid=(NB,),
    in_specs=[_part_spec(16, 0), _part_spec(16, 1),
              pl.BlockSpec((1, RB, 16), lambda i: (0, i, 0)),
              pl.BlockSpec((1, RB, 16), lambda i: (1, i, 0)),
              _row_spec(1), _full_spec((32, 32)), _full_spec((1, 32))],
    out_specs=_row_spec(32),
    out_shape=jax.ShapeDtypeStruct((NP, 32), f32))

_divide_call = pl.pallas_call(
    _tc_divide,
    in_specs=[pl.BlockSpec((2 * GP, 32), lambda: (0, 0)),
              pl.BlockSpec((GP, 1), lambda: (0, 0))],
    out_specs=pl.BlockSpec((G, 32), lambda: (0, 0)),
    out_shape=jax.ShapeDtypeStruct((G, 32), f32))


def kernel(x, edge_index, batch, W1, b1, W2, b2):
    row = edge_index[0].astype(i32)
    col = edge_index[1].astype(i32)
    e = row.shape[0]
    pad = EP - e
    rows_p = jnp.concatenate(
        [row, jnp.zeros((pad,), i32)]).reshape(EP // 128, 128)
    pad_col = N + (jnp.arange(pad, dtype=i32) % (NP - N))
    cols_p = jnp.concatenate([col, pad_col]).reshape(EP // 128, 128)
    pad_batch = G + (jnp.arange(NP - N, dtype=i32) % (GP - G))
    batch_p = jnp.concatenate([batch.astype(i32), pad_batch])
    x_p = jnp.pad(x, ((0, NP - N), (0, 0)))

    zdeg = jnp.zeros((NP,), f32)
    z16 = jnp.zeros((NP, 16), f32)
    zpool = jnp.zeros((GP, 32), f32)

    deg, cnt = _sc_deg_cnt(cols_p, batch_p, zdeg)
    deg_r = _reduce_deg_call(deg.reshape(NW, NP)).reshape(NP, 1)
    cnt_r = _reduce_cnt_call(cnt.reshape(NW, GP)).reshape(GP, 1)
    dinv, gx, tab4 = _prep_call(deg_r, x_p)
    rows_f = rows_p.reshape(EP)
    cols_f = cols_p.reshape(EP)
    a1 = _sc_edge_l1(rows_f, cols_f, tab4.reshape(4 * NP, 16), z16)
    u4 = _l1sum_call(*([a1] * 8), gx.reshape(NP4, 16))
    g2 = _layer1_call(u4.reshape(NP, 4), dinv, W1, b1.reshape(1, 32))
    g2f = g2.reshape(2 * NP, 16)
    a2 = _sc_edge_l2(rows_f, cols_f, g2f, z16)
    out2 = _layer2_call(a2, a2, g2, g2, dinv, W2, b2.reshape(1, 32))
    pools = _sc_pool(out2, batch_p, zpool)
    return _divide_call(pools, cnt_r)


# speedup vs baseline: 16.1494x; 1.0036x over previous
import functools

import jax
import jax.numpy as jnp
from jax import lax
from jax.experimental import pallas as pl
from jax.experimental.pallas import tpu as pltpu
from jax.experimental.pallas import tpu_sc as plsc

N = 100000
G = 2048
NP = 100352
GP = 2304
NC, NS = 2, 16
NW = NC * NS
EP = 1605632
SL = NP // NS
GSL = GP // NS
RB = 3136
NB = NP // RB
PT = NP // NW

_mesh = plsc.VectorSubcoreMesh(
    core_axis_name="c", subcore_axis_name="s", num_cores=NC, num_subcores=NS)
_params = pltpu.CompilerParams(use_tc_tiling_on_sc=False,
                               needs_layout_passes=False)

f32 = jnp.float32
i32 = jnp.int32


@functools.partial(
    pl.kernel,
    out_type=(jax.ShapeDtypeStruct((NW * NP,), f32),
              jax.ShapeDtypeStruct((NW * GP,), f32)),
    mesh=_mesh, compiler_params=_params,
    scratch_types=[pltpu.VMEM((NP,), f32),
                   pltpu.VMEM((GP,), f32),
                   pltpu.VMEM((8, 128), i32),
                   pltpu.VMEM((PT,), i32)])
def _sc_deg_cnt(cols_hbm, batch_hbm, zdeg_hbm,
                deg_out, cnt_out,
                dacc, cacc, cbuf, bbuf):
    c = lax.axis_index("c")
    s = lax.axis_index("s")
    wid = c * NS + s
    pltpu.sync_copy(zdeg_hbm, dacc)
    for t in range(GP // 16):
        cacc[pl.ds(t * 16, 16)] = jnp.zeros((16,), f32)

    ones = jnp.ones((16,), f32)
    iota = lax.iota(i32, 16)
    masks = [iota == k for k in range(16)]

    cpt = (EP // NW) // 128
    base = wid * cpt

    def body(g, carry):
        pltpu.sync_copy(cols_hbm.at[pl.ds(base + g * 8, 8)], cbuf)
        for j in range(8):
            for t in range(8):
                v = cbuf[j, pl.ds(t * 16, 16)]
                for k in range(16):
                    plsc.addupdate_scatter(dacc, [v], ones, mask=masks[k])
        return carry

    lax.fori_loop(0, cpt // 8, body, 0)

    pltpu.sync_copy(batch_hbm.at[pl.ds(wid * PT, PT)], bbuf)

    def bbody(t, carry):
        v = bbuf[pl.ds(t * 16, 16)]
        for k in range(16):
            plsc.addupdate_scatter(cacc, [v], ones, mask=masks[k])
        return carry

    lax.fori_loop(0, PT // 16, bbody, 0)

    pltpu.sync_copy(dacc, deg_out.at[pl.ds(wid * NP, NP)])
    pltpu.sync_copy(cacc, cnt_out.at[pl.ds(wid * GP, GP)])


def _edge_body(rows_hbm, cols_hbm, tab_hbm, z16_hbm, out_hbm,
               acc, ibuf, gbuf, sem, *, cpt, gch, split_edges,
               add_core_off):
    c = lax.axis_index("c")
    s = lax.axis_index("s")
    wid = c * NS + s
    sl = pl.ds(s * SL, SL)
    pltpu.sync_copy(z16_hbm.at[sl], acc.at[sl])
    plsc.subcore_barrier()

    base = (wid if split_edges else s) * cpt * 128
    off = c * NP

    def body(g, carry):
        gb = base + g * gch * 128
        pltpu.sync_copy(rows_hbm.at[pl.ds(gb, gch * 128)], ibuf)
        if add_core_off:
            for t in range(gch * 8):
                ibuf[pl.ds(t * 16, 16)] = ibuf[pl.ds(t * 16, 16)] + off
        pltpu.async_copy(tab_hbm.at[ibuf], gbuf, sem).wait()
        pltpu.sync_copy(cols_hbm.at[pl.ds(gb, gch * 128)], ibuf)
        for r in range(NS):
            @pl.when(s == r)
            def _():
                pltpu.sync_copy(gbuf, acc.at[ibuf], add=True)
            plsc.subcore_barrier()
        return carry

    lax.fori_loop(0, cpt // gch, body, 0)
    plsc.subcore_barrier()
    pltpu.sync_copy(acc.at[sl], out_hbm.at[pl.ds(c * NP + s * SL, SL)])


def _make_edge_pass(*, cpt, gch, split_edges, add_core_off):
    @functools.partial(
        pl.kernel,
        out_type=jax.ShapeDtypeStruct((2 * NP, 16), f32),
        mesh=_mesh, compiler_params=_params,
        scratch_types=[pltpu.VMEM_SHARED((NP, 16), f32),
                       pltpu.VMEM((gch * 128,), i32),
                       pltpu.VMEM((gch * 128, 16), f32),
                       pltpu.SemaphoreType.DMA])
    def k(rows_hbm, cols_hbm, tab_hbm, z16_hbm, out_hbm,
          acc, ibuf, gbuf, sem):
        _edge_body(rows_hbm, cols_hbm, tab_hbm, z16_hbm, out_hbm,
                   acc, ibuf, gbuf, sem, cpt=cpt, gch=gch,
                   split_edges=split_edges, add_core_off=add_core_off)
    return k


_sc_edge_l2 = _make_edge_pass(cpt=(EP // NS) // 128, gch=14,
                              split_edges=False, add_core_off=True)

NP4 = NP // 4


@functools.partial(
    pl.kernel,
    out_type=jax.ShapeDtypeStruct((2 * NP, 16), f32),
    mesh=_mesh, compiler_params=_params,
    scratch_types=[pltpu.VMEM_SHARED((NP, 16), f32),
                   pltpu.VMEM((1024,), i32),
                   pltpu.VMEM((1024,), i32),
                   pltpu.VMEM((1024, 16), f32),
                   pltpu.SemaphoreType.DMA])
def _sc_edge_l1(rows_hbm, cols_hbm, tab4_hbm, z16_hbm, out_hbm,
                acc, rbuf, cbuf, gbuf, sem):
    c = lax.axis_index("c")
    s = lax.axis_index("s")
    wid = c * NS + s
    sl = pl.ds(s * SL, SL)
    pltpu.sync_copy(z16_hbm.at[sl], acc.at[sl])
    plsc.subcore_barrier()

    cpt = (EP // NW) // 128
    base = wid * cpt * 128
    qbase = (s // 4) * NP4

    def body(g, carry):
        gb = base + g * 1024
        pltpu.sync_copy(rows_hbm.at[pl.ds(gb, 1024)], rbuf)
        pltpu.sync_copy(cols_hbm.at[pl.ds(gb, 1024)], cbuf)
        for t in range(64):
            dst = pl.ds(t * 16, 16)
            rv = rbuf[dst]
            cv = cbuf[dst]
            rbuf[dst] = rv * 4 + jnp.bitwise_and(cv, 3)
            cbuf[dst] = lax.shift_right_logical(cv, 2) + qbase
        pltpu.async_copy(tab4_hbm.at[rbuf], gbuf, sem).wait()
        for r in range(4):
            @pl.when(s % 4 == r)
            def _():
                pltpu.sync_copy(gbuf, acc.at[cbuf], add=True)
            plsc.subcore_barrier()
        return carry

    lax.fori_loop(0, cpt // 8, body, 0)
    plsc.subcore_barrier()
    pltpu.sync_copy(acc.at[sl], out_hbm.at[pl.ds(c * NP + s * SL, SL)])


@functools.partial(
    pl.kernel,
    out_type=jax.ShapeDtypeStruct((2 * GP, 32), f32),
    mesh=_mesh, compiler_params=_params,
    scratch_types=[pltpu.VMEM_SHARED((GP, 32), f32),
                   pltpu.VMEM((448,), i32),
                   pltpu.VMEM((448, 32), f32),
                   pltpu.VMEM((GSL, 32), f32),
                   pltpu.SemaphoreType.DMA])
def _sc_pool(out2_hbm, batchf_hbm, zpool_hbm,
             pool_out,
             acc, bbuf, robuf, vbuf, sem):
    c = lax.axis_index("c")
    s = lax.axis_index("s")
    wid = c * NS + s
    gsl = pl.ds(s * GSL, GSL)
    pltpu.sync_copy(zpool_hbm.at[gsl], acc.at[gsl])
    plsc.subcore_barrier()

    base = wid * PT
    for g in range(7):
        pltpu.sync_copy(
            batchf_hbm.at[pl.ds(base + g * 448, 448)], bbuf)
        pltpu.async_copy(
            out2_hbm.at[pl.ds(base + g * 448, 448)], robuf, sem).wait()
        for r in range(NS):
            @pl.when(s == r)
            def _():
                pltpu.sync_copy(robuf, acc.at[bbuf], add=True)
            plsc.subcore_barrier()

    pltpu.sync_copy(acc.at[gsl], vbuf)
    pltpu.sync_copy(vbuf, pool_out.at[pl.ds(c * GP + s * GSL, GSL)])


def _tc_sum_parts(deg, cnt, dout, cout):
    dout[...] = jnp.sum(deg[...], axis=0, keepdims=True)

    @pl.when(pl.program_id(0) == 0)
    def _():
        cout[...] = jnp.sum(cnt[...], axis=0, keepdims=True)


def _tc_prep(deg, x, dinv, gx, tab4):
    dv = lax.rsqrt(deg[...] + 1.0)
    dinv[...] = dv
    g = x[...] * dv
    gx[...] = g
    z = jnp.zeros((RB, 4), f32)
    tab4[:, 0, :] = jnp.concatenate([g, z, z, z], axis=1)
    tab4[:, 1, :] = jnp.concatenate([z, g, z, z], axis=1)
    tab4[:, 2, :] = jnp.concatenate([z, z, g, z], axis=1)
    tab4[:, 3, :] = jnp.concatenate([z, z, z, g], axis=1)


def _tc_l1sum(*refs):
    parts = refs[:8]
    gx4 = refs[8]
    u4 = refs[9]
    acc = gx4[...]
    for p in parts:
        acc = acc + p[...]
    u4[...] = acc


def _tc_layer1(u, dinv, w1, b1, g2):
    dv = dinv[...]
    p1 = dv * u[...]
    x1 = jnp.maximum(
        jnp.dot(p1, w1[...], preferred_element_type=f32) + b1[...], 0.0)
    h = x1 * dv
    g2[0] = h[:, :16]
    g2[1] = h[:, 16:]


def _tc_layer2(a2a, a2b, g2a, g2b, dinv, w2, b2, out2):
    dv = dinv[...]
    pa = dv * (a2a[...] + g2a[0])
    pb = dv * (a2b[...] + g2b[0])
    p2 = jnp.concatenate([pa, pb], axis=1)
    out2[...] = jnp.maximum(
        jnp.dot(p2, w2[...], preferred_element_type=f32) + b2[...], 0.0)


def _tc_divide(pools, cnts, out):
    pv = pools[...]
    cv = cnts[...]
    p = pv[0:G] + pv[GP:GP + G]
    out[...] = p / jnp.maximum(cv[0:G], 1.0)


def _part_spec(width, part):
    return pl.BlockSpec((RB, width), lambda i, p=part: (i + p * NB, 0))


def _row_spec(width):
    return pl.BlockSpec((RB, width), lambda i: (i, 0))


def _full_spec(shape):
    return pl.BlockSpec(shape, lambda i: tuple(0 for _ in shape))


_reduce_call = pl.pallas_call(
    _tc_sum_parts,
    grid=(NP // 6272,),
    in_specs=[pl.BlockSpec((NW, 6272), lambda i: (0, i)),
              pl.BlockSpec((NW, GP), lambda i: (0, 0))],
    out_specs=[pl.BlockSpec((1, 6272), lambda i: (0, i)),
               pl.BlockSpec((1, GP), lambda i: (0, 0))],
    out_shape=(jax.ShapeDtypeStruct((1, NP), f32),
               jax.ShapeDtypeStruct((1, GP), f32)))

_prep_call = pl.pallas_call(
    _tc_prep,
    grid=(NB,),
    in_specs=[_row_spec(1), _row_spec(4)],
    out_specs=[_row_spec(1), _row_spec(4),
               pl.BlockSpec((RB, 4, 16), lambda i: (i, 0, 0))],
    out_shape=(jax.ShapeDtypeStruct((NP, 1), f32),
               jax.ShapeDtypeStruct((NP, 4), f32),
               jax.ShapeDtypeStruct((NP, 4, 16), f32)))

_l1sum_call = pl.pallas_call(
    _tc_l1sum,
    grid=(NP4 // RB,),
    in_specs=([pl.BlockSpec((RB, 16), lambda i, q=q: (i + q * (NP4 // RB), 0))
               for q in range(4)] +
              [pl.BlockSpec((RB, 16),
                            lambda i, q=q: (i + NB + q * (NP4 // RB), 0))
               for q in range(4)] +
              [pl.BlockSpec((RB, 16), lambda i: (i, 0))]),
    out_specs=pl.BlockSpec((RB, 16), lambda i: (i, 0)),
    out_shape=jax.ShapeDtypeStruct((NP4, 16), f32))

_layer1_call = pl.pallas_call(
    _tc_layer1,
    grid=(NB,),
    in_specs=[_row_spec(4),
              _row_spec(1), _full_spec((4, 32)), _full_spec((1, 32))],
    out_specs=pl.BlockSpec((2, RB, 16), lambda i: (0, i, 0)),
    out_shape=jax.ShapeDtypeStruct((2, NP, 16), f32))

_layer2_call = pl.pallas_call(
    _tc_layer2,
    grid=(NB,),
    in_specs=[_part_spec(16, 0), _part_spec(16, 1),
              pl.BlockSpec((1, RB, 16), lambda i: (0, i, 0)),
              pl.BlockSpec((1, RB, 16), lambda i: (1, i, 0)),
              _row_spec(1), _full_spec((32, 32)), _full_spec((1, 32))],
    out_specs=_row_spec(32),
    out_shape=jax.ShapeDtypeStruct((NP, 32), f32))

_divide_call = pl.pallas_call(
    _tc_divide,
    in_specs=[pl.BlockSpec((2 * GP, 32), lambda: (0, 0)),
              pl.BlockSpec((GP, 1), lambda: (0, 0))],
    out_specs=pl.BlockSpec((G, 32), lambda: (0, 0)),
    out_shape=jax.ShapeDtypeStruct((G, 32), f32))


def kernel(x, edge_index, batch, W1, b1, W2, b2):
    row = edge_index[0].astype(i32)
    col = edge_index[1].astype(i32)
    e = row.shape[0]
    pad = EP - e
    rows_p = jnp.concatenate(
        [row, jnp.zeros((pad,), i32)]).reshape(EP // 128, 128)
    pad_col = N + (jnp.arange(pad, dtype=i32) % (NP - N))
    cols_p = jnp.concatenate([col, pad_col]).reshape(EP // 128, 128)
    pad_batch = G + (jnp.arange(NP - N, dtype=i32) % (GP - G))
    batch_p = jnp.concatenate([batch.astype(i32), pad_batch])
    x_p = jnp.pad(x, ((0, NP - N), (0, 0)))

    zdeg = jnp.zeros((NP,), f32)
    z16 = jnp.zeros((NP, 16), f32)
    zpool = jnp.zeros((GP, 32), f32)

    deg, cnt = _sc_deg_cnt(cols_p, batch_p, zdeg)
    deg_r, cnt_r = _reduce_call(deg.reshape(NW, NP), cnt.reshape(NW, GP))
    deg_r = deg_r.reshape(NP, 1)
    cnt_r = cnt_r.reshape(GP, 1)
    dinv, gx, tab4 = _prep_call(deg_r, x_p)
    rows_f = rows_p.reshape(EP)
    cols_f = cols_p.reshape(EP)
    a1 = _sc_edge_l1(rows_f, cols_f, tab4.reshape(4 * NP, 16), z16)
    u4 = _l1sum_call(*([a1] * 8), gx.reshape(NP4, 16))
    g2 = _layer1_call(u4.reshape(NP, 4), dinv, W1, b1.reshape(1, 32))
    g2f = g2.reshape(2 * NP, 16)
    a2 = _sc_edge_l2(rows_f, cols_f, g2f, z16)
    out2 = _layer2_call(a2, a2, g2, g2, dinv, W2, b2.reshape(1, 32))
    pools = _sc_pool(out2, batch_p, zpool)
    return _divide_call(pools, cnt_r)
